# Initial kernel scaffold; baseline (speedup 1.0000x reference)
import jax
import jax.numpy as jnp
from jax.experimental import pallas as pl


def kernel(probabilities):
    raise NotImplementedError("write your pallas kernel here")


# speedup vs baseline: 6.7679x; 6.7679x over previous
import functools

import jax
import jax.numpy as jnp
from jax import lax
from jax.experimental import pallas as pl
from jax.experimental.pallas import tpu as pltpu
import jax.experimental.pallas.tpu_sc as plsc

L = 16
NW = 16
SB = 256
NSB = 245
CHUNK = SB * NSB
NPAD = CHUNK * NW


def _splat_max(v):
    return jnp.max(v)


def _body(p_hbm, c_hbm, out_hbm, data_v, c_v, lsum_v, parts_v, cnt_v, shared_s):
    w = lax.axis_index("s") * 1 + lax.axis_index("c")

    pltpu.sync_copy(p_hbm.at[pl.ds(w * CHUNK, CHUNK)], data_v)
    pltpu.sync_copy(c_hbm, c_v)

    def sum_body(i, acc):
        base = i * SB
        for k in range(SB // L):
            acc = acc + data_v[pl.ds(base + k * L, L)]
        return acc

    acc = lax.fori_loop(0, NSB, sum_body, jnp.zeros((L,), jnp.float32))
    local = jnp.sum(acc)

    lsum_v[...] = jnp.full((L,), local, jnp.float32)
    pltpu.sync_copy(lsum_v, shared_s.at[pl.ds(w * L, L)])
    plsc.subcore_barrier()
    pltpu.sync_copy(shared_s, parts_v)

    total_vec = jnp.zeros((L,), jnp.float32)
    prefix_vec = jnp.zeros((L,), jnp.float32)
    for i in range(NW):
        row = parts_v[pl.ds(i * L, L)]
        total_vec = total_vec + row
        prefix_vec = prefix_vec + jnp.where(i < w, row, jnp.zeros((L,), jnp.float32))
    total = _splat_max(total_vec)
    prefix = _splat_max(prefix_vec)

    u = _splat_max(c_v[...]) * total
    r = u - prefix

    def sb_body(i, carry):
        run, cnt, bd_j, bd_run = carry
        acc = jnp.zeros((L,), jnp.float32)
        base = i * SB
        for k in range(SB // L):
            acc = acc + data_v[pl.ds(base + k * L, L)]
        s = jnp.sum(acc)
        incl = run + s
        below = incl <= r
        cross = jnp.logical_and(run <= r, incl > r)
        cnt = cnt + jnp.where(below, SB, 0)
        bd_j = jnp.where(cross, i, bd_j)
        bd_run = jnp.where(cross, run, bd_run)
        return incl, cnt, bd_j, bd_run

    init = (jnp.float32(0.0), jnp.int32(0), jnp.int32(NSB), jnp.float32(0.0))
    _, cnt, bd_j, bd_run = lax.fori_loop(0, NSB, sb_body, init)

    bd_jc = jnp.minimum(bd_j, NSB - 1)
    base = bd_jc * SB
    fcnt_vec = jnp.zeros((L,), jnp.int32)
    fc = bd_run
    for k in range(SB // L):
        v = data_v[pl.ds(base + k * L, L)]
        cs = plsc.cumsum(v) + fc
        m = cs <= r
        fcnt_vec = fcnt_vec + plsc.all_reduce_population_count(m)
        fc = fc + jnp.sum(v)
    fcnt = jnp.max(fcnt_vec)

    count = cnt + jnp.where(bd_j < NSB, fcnt, 0)
    cnt_v[...] = jnp.full((L,), count, jnp.int32)
    pltpu.sync_copy(cnt_v, out_hbm.at[pl.ds(w * L, L)])


@jax.jit
def _sc_count(p_pad, c_vec):
    mesh = plsc.VectorSubcoreMesh(
        core_axis_name="c", subcore_axis_name="s", num_cores=1, num_subcores=NW
    )
    f = pl.kernel(
        _body,
        out_type=jax.ShapeDtypeStruct((NW * L,), jnp.int32),
        mesh=mesh,
        compiler_params=pltpu.CompilerParams(needs_layout_passes=False),
        scratch_types=[
            pltpu.VMEM((CHUNK,), jnp.float32),
            pltpu.VMEM((L,), jnp.float32),
            pltpu.VMEM((L,), jnp.float32),
            pltpu.VMEM((NW * L,), jnp.float32),
            pltpu.VMEM((L,), jnp.int32),
            pltpu.VMEM_SHARED((NW * L,), jnp.float32),
        ],
    )
    return f(p_pad, c_vec)


def kernel(probabilities):
    n = probabilities.shape[0]
    c = jax.random.uniform(jax.random.key(42), (), dtype=jnp.float32)
    p_pad = jnp.pad(probabilities, (0, NPAD - n))
    counts = _sc_count(p_pad, jnp.full((L,), c, jnp.float32))
    idx = jnp.sum(counts.reshape(NW, L)[:, 0])
    return jnp.minimum(idx, n - 1).astype(jnp.int32)
